# baseline (speedup 1.0000x reference)
import functools

import jax
import jax.numpy as jnp
from jax import lax
from jax.experimental import pallas as pl
from jax.experimental.pallas import tpu as pltpu
from jax.experimental.pallas import tpu_sc as plsc

N = 10000
E = 320000
D = 128
DE = 16
H = 64

NC = 2
NS = 16
CHUNK = 64
EPT = 20480
E_PAD = EPT * NS
NSTEP = EPT // CHUNK
N_ACC = 10112
ROWS_PER_TILE = N_ACC // NS

F32 = jnp.float32


_RB = 2000
_NB = N // _RB


def _k1_body(x_ref, w_ref, b_ref, h_ref, s_ref, ss_ref):
  h = jnp.dot(x_ref[...], w_ref[...], preferred_element_type=F32) + b_ref[...]
  h_ref[...] = h
  s_ref[...] = jnp.sum(h, axis=0, keepdims=True)[None]
  ss_ref[...] = jnp.sum(h * h, axis=0, keepdims=True)[None]


_k1 = pl.pallas_call(
    _k1_body,
    grid=(_NB,),
    in_specs=[
        pl.BlockSpec((_RB, D), lambda i: (i, 0)),
        pl.BlockSpec((D, D), lambda i: (0, 0)),
        pl.BlockSpec((1, D), lambda i: (0, 0)),
    ],
    out_specs=[
        pl.BlockSpec((_RB, D), lambda i: (i, 0)),
        pl.BlockSpec((1, 1, D), lambda i: (i, 0, 0)),
        pl.BlockSpec((1, 1, D), lambda i: (i, 0, 0)),
    ],
    out_shape=[
        jax.ShapeDtypeStruct((N, D), F32),
        jax.ShapeDtypeStruct((_NB, 1, D), F32),
        jax.ShapeDtypeStruct((_NB, 1, D), F32),
    ],
)


def _k2_first_body(h_ref, s_ref, ss_ref, g_ref, be_ref, fw_ref, fb_ref,
                   fwh_ref, fbh_ref, wd_ref, wsh_ref, src2_ref, pd_ref):
  mean = jnp.sum(s_ref[...], axis=0) / N
  var = jnp.sum(ss_ref[...], axis=0) / N - mean * mean
  hn = (h_ref[...] - mean) * (g_ref[...] * lax.rsqrt(var + 1e-5)) + be_ref[...]
  t = jnp.dot(hn, fw_ref[...], preferred_element_type=F32) + fb_ref[...]
  ps_h = jnp.dot(t, wsh_ref[0], preferred_element_type=F32)
  t_h = jnp.dot(hn, fwh_ref[0], preferred_element_type=F32) + fbh_ref[0]
  src2_ref[0] = jnp.concatenate([ps_h, t_h], axis=1)
  pd_ref[...] = jnp.dot(t, wd_ref[...], preferred_element_type=F32)


_k2_first = pl.pallas_call(
    _k2_first_body,
    grid=(_NB, NC),
    in_specs=[
        pl.BlockSpec((_RB, D), lambda i, c: (i, 0)),
        pl.BlockSpec((_NB, 1, D), lambda i, c: (0, 0, 0)),
        pl.BlockSpec((_NB, 1, D), lambda i, c: (0, 0, 0)),
        pl.BlockSpec((1, D), lambda i, c: (0, 0)),
        pl.BlockSpec((1, D), lambda i, c: (0, 0)),
        pl.BlockSpec((D, D), lambda i, c: (0, 0)),
        pl.BlockSpec((1, D), lambda i, c: (0, 0)),
        pl.BlockSpec((1, D, H), lambda i, c: (c, 0, 0)),
        pl.BlockSpec((1, 1, H), lambda i, c: (c, 0, 0)),
        pl.BlockSpec((D, D), lambda i, c: (0, 0)),
        pl.BlockSpec((1, D, H), lambda i, c: (c, 0, 0)),
    ],
    out_specs=[
        pl.BlockSpec((1, _RB, D), lambda i, c: (c, i, 0)),
        pl.BlockSpec((_RB, D), lambda i, c: (i, 0)),
    ],
    out_shape=[
        jax.ShapeDtypeStruct((NC, N, D), F32),
        jax.ShapeDtypeStruct((N, D), F32),
    ],
)


def _k2_mid_body(acc_ref, fw_ref, fb_ref, fwh_ref, fbh_ref, wd_ref, wsh_ref,
                 src2_ref, pd_ref):
  a0, a1 = acc_ref[0], acc_ref[1]
  h = jnp.concatenate(
      [a0[:, H:] / (a0[:, :H] + 1e-16), a1[:, H:] / (a1[:, :H] + 1e-16)],
      axis=1)
  t = jnp.dot(h, fw_ref[...], preferred_element_type=F32) + fb_ref[...]
  ps_h = jnp.dot(t, wsh_ref[0], preferred_element_type=F32)
  t_h = jnp.dot(h, fwh_ref[0], preferred_element_type=F32) + fbh_ref[0]
  src2_ref[0] = jnp.concatenate([ps_h, t_h], axis=1)
  pd_ref[...] = jnp.dot(t, wd_ref[...], preferred_element_type=F32)


_k2_mid = pl.pallas_call(
    _k2_mid_body,
    grid=(_NB, NC),
    in_specs=[
        pl.BlockSpec((NC, _RB, D), lambda i, c: (0, i, 0)),
        pl.BlockSpec((D, D), lambda i, c: (0, 0)),
        pl.BlockSpec((1, D), lambda i, c: (0, 0)),
        pl.BlockSpec((1, D, H), lambda i, c: (c, 0, 0)),
        pl.BlockSpec((1, 1, H), lambda i, c: (c, 0, 0)),
        pl.BlockSpec((D, D), lambda i, c: (0, 0)),
        pl.BlockSpec((1, D, H), lambda i, c: (c, 0, 0)),
    ],
    out_specs=[
        pl.BlockSpec((1, _RB, D), lambda i, c: (c, i, 0)),
        pl.BlockSpec((_RB, D), lambda i, c: (i, 0)),
    ],
    out_shape=[
        jax.ShapeDtypeStruct((NC, N, D), F32),
        jax.ShapeDtypeStruct((N, D), F32),
    ],
)


_EB = 1024
_NEB = E_PAD // 2 // _EB


def _k3_body(eae_ref, eao_ref, we_ref, be_ref, wa_ref, ab_ref, pe_ref):
  i = pl.program_id(0)
  wa = wa_ref[0]
  m = jnp.dot(we_ref[...], wa, preferred_element_type=F32)
  cvec = (jnp.dot(be_ref[...], wa, preferred_element_type=F32)
          + ab_ref[0])
  pe_even = jnp.dot(eae_ref[...], m, preferred_element_type=F32) + cvec
  pe_odd = jnp.dot(eao_ref[...], m, preferred_element_type=F32) + cvec
  row = i * _EB + lax.broadcasted_iota(jnp.int32, (_EB, 1), 0)
  ok = row < E // 2
  pe_ref[0] = jnp.concatenate(
      [jnp.where(ok, pe_even, -1e30), jnp.where(ok, pe_odd, -1e30)], axis=1)


_k3 = pl.pallas_call(
    _k3_body,
    grid=(_NEB, NC),
    in_specs=[
        pl.BlockSpec((_EB, DE), lambda i, c: (i, 0)),
        pl.BlockSpec((_EB, DE), lambda i, c: (i, 0)),
        pl.BlockSpec((DE, DE), lambda i, c: (0, 0)),
        pl.BlockSpec((1, DE), lambda i, c: (0, 0)),
        pl.BlockSpec((1, DE, H), lambda i, c: (c, 0, 0)),
        pl.BlockSpec((1, 1, H), lambda i, c: (c, 0, 0)),
    ],
    out_specs=pl.BlockSpec((1, _EB, D), lambda i, c: (c, i, 0)),
    out_shape=jax.ShapeDtypeStruct((NC, E_PAD // 2, D), F32),
)


def _k4_body(acc_ref, out_ref):
  a0, a1 = acc_ref[0], acc_ref[1]
  out_ref[...] = jnp.concatenate(
      [a0[:, H:] / (a0[:, :H] + 1e-16), a1[:, H:] / (a1[:, :H] + 1e-16)],
      axis=1)


_k4 = pl.pallas_call(
    _k4_body,
    grid=(_NB,),
    in_specs=[
        pl.BlockSpec((NC, _RB, D), lambda i: (0, i, 0)),
    ],
    out_specs=pl.BlockSpec((_RB, D), lambda i: (i, 0)),
    out_shape=jax.ShapeDtypeStruct((N, D), F32),
)


@functools.cache
def _build_sc_edge_pass():
  sc_mesh = plsc.VectorSubcoreMesh(
      core_axis_name="c", subcore_axis_name="s", num_cores=NC,
      num_subcores=NS)

  @functools.partial(
      pl.kernel,
      out_type=jax.ShapeDtypeStruct((NC * N_ACC, D), F32),
      mesh=sc_mesh,
      scratch_types=[
          pltpu.VMEM((CHUNK,), jnp.int32),
          pltpu.VMEM((CHUNK,), jnp.int32),
          pltpu.VMEM((CHUNK,), jnp.int32),
          pltpu.VMEM((CHUNK,), jnp.int32),
          pltpu.VMEM((CHUNK, D), F32),
          pltpu.VMEM((CHUNK, D), F32),
          pltpu.VMEM((CHUNK // 2, D), F32),
          pltpu.VMEM((CHUNK, D), F32),
          pltpu.VMEM((CHUNK, D), F32),
          pltpu.VMEM((CHUNK // 2, D), F32),
          pltpu.VMEM((CHUNK,), jnp.int32),
          pltpu.VMEM((CHUNK,), jnp.int32),
          pltpu.VMEM_SHARED((N_ACC, D), F32),
          pltpu.SemaphoreType.DMA,
          pltpu.SemaphoreType.DMA,
          pltpu.SemaphoreType.DMA,
          pltpu.SemaphoreType.DMA,
          pltpu.SemaphoreType.DMA,
          pltpu.SemaphoreType.DMA,
      ],
  )
  def sc_edge_pass(src_hbm, dst_hbm, src2_hbm, pd_hbm, pe_hbm,
                   acc_hbm,
                   isa, ida, isb, idb, sra, pda, pea, srb, pdb, peb,
                   sda, sdb, acc, sema, semb, isema, isemb, ssema, ssemb):
    c = lax.axis_index("c")
    s = lax.axis_index("s")
    cH = c * H
    set_a = (isa, ida, sra, pda, pea, sema, isema, sda, ssema)
    set_b = (isb, idb, srb, pdb, peb, semb, isemb, sdb, ssemb)

    zero = jnp.zeros((16,), F32)
    for r in range(8):
      for j in range(D // 16):
        sra[r, pl.ds(j * 16, 16)] = zero

    def zinit(k, carry):
      pltpu.sync_copy(sra.at[pl.ds(0, 8)],
                      acc.at[pl.ds(s * ROWS_PER_TILE + k * 8, 8)])
      return carry

    lax.fori_loop(0, ROWS_PER_TILE // 8, zinit, 0)
    plsc.subcore_barrier()

    def issue_idx(g, bufs):
      i_s, i_d = bufs[0], bufs[1]
      isem = bufs[6]
      base = s * EPT + g * CHUNK
      pltpu.async_copy(src_hbm.at[pl.ds(base, CHUNK)], i_s, isem)
      pltpu.async_copy(dst_hbm.at[pl.ds(base, CHUNK)], i_d, isem)

    def prefetch(g, bufs):
      i_s, i_d, sr, pd_, pe_, sem, isem = bufs[:7]
      base = s * EPT + g * CHUNK
      pltpu.make_async_copy(src_hbm.at[pl.ds(0, CHUNK)], i_s, isem).wait()
      pltpu.make_async_copy(dst_hbm.at[pl.ds(0, CHUNK)], i_d, isem).wait()
      for j in range(CHUNK // 16):
        sl = pl.ds(j * 16, 16)
        i_s[sl] = i_s[sl] + c * N
      pltpu.async_copy(src2_hbm.at[i_s], sr, sem)
      pltpu.async_copy(pd_hbm.at[i_d], pd_, sem)
      base2 = s * (EPT // 2) + g * (CHUNK // 2)
      pltpu.async_copy(pe_hbm.at[pl.ds(c * (E_PAD // 2) + base2, CHUNK // 2)],
                       pe_, sem)

    def wait_set(bufs):
      i_s, i_d, sr, pd_, pe_, sem = bufs[:6]
      pltpu.make_async_copy(src2_hbm.at[i_s], sr, sem).wait()
      pltpu.make_async_copy(pd_hbm.at[i_d], pd_, sem).wait()
      pltpu.make_async_copy(pe_hbm.at[pl.ds(0, CHUNK // 2)], pe_, sem).wait()

    def wait_scatter(bufs):
      pd_, sd, ssem = bufs[3], bufs[7], bufs[8]
      pltpu.make_async_copy(pd_, acc.at[sd], ssem).wait()

    def compute_scatter(bufs):
      i_s, i_d, sr, pd_, pe_, sem, _, sd, ssem = bufs

      @plsc.parallel_loop(0, CHUNK // 2, step=1, unroll=2)
      def edge_body(p):
        for r in range(2):
          e = 2 * p + r
          for j in range(H // 16):
            slh = pl.ds(cH + j * 16, 16)
            sl = pl.ds(j * 16, 16)
            slt = pl.ds(H + j * 16, 16)
            a = pd_[e, slh] + sr[e, sl] + pe_[p, pl.ds(r * H + j * 16, 16)]
            a = jnp.maximum(a, a * 0.2)
            w = jnp.exp(a)
            tv = sr[e, slt]
            pd_[e, sl] = w
            pd_[e, slt] = w * tv
      for j in range(CHUNK // 16):
        sl = pl.ds(j * 16, 16)
        sd[sl] = i_d[sl]
      pltpu.async_copy(pd_, acc.at[sd], ssem, add=True)

    def pair_body(i, carry):
      g0 = 2 * i
      wait_set(set_a)

      @pl.when(i > 0)
      def _():
        wait_scatter(set_b)

      prefetch(g0 + 1, set_b)
      compute_scatter(set_a)

      @pl.when(g0 + 2 < NSTEP)
      def _():
        issue_idx(g0 + 2, set_a)

      wait_set(set_b)
      wait_scatter(set_a)

      @pl.when(g0 + 2 < NSTEP)
      def _():
        prefetch(g0 + 2, set_a)

      compute_scatter(set_b)

      @pl.when(g0 + 3 < NSTEP)
      def _():
        issue_idx(g0 + 3, set_b)

      return carry

    issue_idx(0, set_a)
    prefetch(0, set_a)
    issue_idx(1, set_b)
    lax.fori_loop(0, NSTEP // 2, pair_body, 0)
    wait_scatter(set_b)
    plsc.subcore_barrier()

    def flush(k, carry):
      r0 = s * ROWS_PER_TILE + k * 64
      pltpu.sync_copy(acc.at[pl.ds(r0, 64)], pdb)
      pltpu.sync_copy(pdb, acc_hbm.at[pl.ds(c * N_ACC + r0, 64)])
      return carry

    lax.fori_loop(0, ROWS_PER_TILE // 64, flush, 0)
    tail = ROWS_PER_TILE % 64
    if tail:
      r0 = s * ROWS_PER_TILE + (ROWS_PER_TILE // 64) * 64
      pltpu.sync_copy(acc.at[pl.ds(r0, tail)], pdb.at[pl.ds(0, tail)])
      pltpu.sync_copy(pdb.at[pl.ds(0, tail)],
                      acc_hbm.at[pl.ds(c * N_ACC + r0, tail)])

  return sc_edge_pass


def _split_cols(w):
  return w.reshape(D, NC, H).transpose(1, 0, 2)


def kernel(x, edge_index, edge_attr, W_node, b_node, W_edge, b_edge,
           gamma, beta, fc_W, fc_b, attn_W, attn_b):
  src = edge_index[0].astype(jnp.int32)
  dst = edge_index[1].astype(jnp.int32)
  src_pad = jnp.pad(src, (0, E_PAD - E))
  dst_pad = jnp.pad(dst, (0, E_PAD - E))
  ea_pad = jnp.pad(edge_attr.astype(F32), ((0, E_PAD - E), (0, 0)))

  h_pre, psum, psumsq = _k1(x, W_node, b_node.reshape(1, D))

  acc = None
  for l in range(3):
    aW = attn_W[l]
    wd, ws, wa = aW[:D], aW[D:2 * D], aW[2 * D:]
    fw = fc_W[l]
    fwh = _split_cols(fw)
    fbh = fc_b[l].reshape(NC, 1, H)
    wsh = _split_cols(ws)
    if l == 0:
      src2, pd = _k2_first(h_pre, psum, psumsq, gamma.reshape(1, D),
                           beta.reshape(1, D), fw, fc_b[0].reshape(1, D),
                           fwh, fbh, wd, wsh)
    else:
      src2, pd = _k2_mid(acc, fw, fc_b[l].reshape(1, D), fwh, fbh, wd, wsh)
    wa_split = wa.reshape(DE, NC, H).transpose(1, 0, 2)
    pe = _k3(ea_pad[0::2], ea_pad[1::2], W_edge, b_edge.reshape(1, DE),
             wa_split, attn_b[l].reshape(NC, 1, H))
    pe = pe.reshape(NC * (E_PAD // 2), D)
    acc = _build_sc_edge_pass()(
        src_pad, dst_pad, src2.reshape(NC * N, D), pd, pe)
    acc = acc.reshape(NC, N_ACC, D)[:, :N]

  h = _k4(acc)
  return h.reshape(1, N, D)


# speedup vs baseline: 1.3608x; 1.3608x over previous
import functools

import jax
import jax.numpy as jnp
from jax import lax
from jax.experimental import pallas as pl
from jax.experimental.pallas import tpu as pltpu
from jax.experimental.pallas import tpu_sc as plsc

N = 10000
E = 320000
D = 128
DE = 16
H = 64

NC = 2
NS = 16
CHUNK = 64
EPT = 20480
E_PAD = EPT * NS
NSTEP = EPT // CHUNK
N_ACC = 10112
ROWS_PER_TILE = N_ACC // NS

F32 = jnp.float32


_RB = 2000
_NB = N // _RB


def _k1_body(x_ref, w_ref, b_ref, h_ref, s_ref, ss_ref):
  h = jnp.dot(x_ref[...], w_ref[...], preferred_element_type=F32) + b_ref[...]
  h_ref[...] = h
  s_ref[...] = jnp.sum(h, axis=0, keepdims=True)[None]
  ss_ref[...] = jnp.sum(h * h, axis=0, keepdims=True)[None]


_k1 = pl.pallas_call(
    _k1_body,
    grid=(_NB,),
    in_specs=[
        pl.BlockSpec((_RB, D), lambda i: (i, 0)),
        pl.BlockSpec((D, D), lambda i: (0, 0)),
        pl.BlockSpec((1, D), lambda i: (0, 0)),
    ],
    out_specs=[
        pl.BlockSpec((_RB, D), lambda i: (i, 0)),
        pl.BlockSpec((1, 1, D), lambda i: (i, 0, 0)),
        pl.BlockSpec((1, 1, D), lambda i: (i, 0, 0)),
    ],
    out_shape=[
        jax.ShapeDtypeStruct((N, D), F32),
        jax.ShapeDtypeStruct((_NB, 1, D), F32),
        jax.ShapeDtypeStruct((_NB, 1, D), F32),
    ],
)


def _k2_first_body(h_ref, s_ref, ss_ref, g_ref, be_ref, fw_ref, fb_ref,
                   fwh_ref, fbh_ref, wdh_ref, wsh_ref, src2_ref, pd_ref):
  mean = jnp.sum(s_ref[...], axis=0) / N
  var = jnp.sum(ss_ref[...], axis=0) / N - mean * mean
  hn = (h_ref[...] - mean) * (g_ref[...] * lax.rsqrt(var + 1e-5)) + be_ref[...]
  t = jnp.dot(hn, fw_ref[...], preferred_element_type=F32) + fb_ref[...]
  ps_h = jnp.dot(t, wsh_ref[0], preferred_element_type=F32)
  t_h = jnp.dot(hn, fwh_ref[0], preferred_element_type=F32) + fbh_ref[0]
  src2_ref[0] = jnp.concatenate([ps_h, t_h], axis=1)
  pd_h = jnp.dot(t, wdh_ref[0], preferred_element_type=F32)
  pd_ref[0] = jnp.concatenate([pd_h, pd_h], axis=1)


_k2_first = pl.pallas_call(
    _k2_first_body,
    grid=(_NB, NC),
    in_specs=[
        pl.BlockSpec((_RB, D), lambda i, c: (i, 0)),
        pl.BlockSpec((_NB, 1, D), lambda i, c: (0, 0, 0)),
        pl.BlockSpec((_NB, 1, D), lambda i, c: (0, 0, 0)),
        pl.BlockSpec((1, D), lambda i, c: (0, 0)),
        pl.BlockSpec((1, D), lambda i, c: (0, 0)),
        pl.BlockSpec((D, D), lambda i, c: (0, 0)),
        pl.BlockSpec((1, D), lambda i, c: (0, 0)),
        pl.BlockSpec((1, D, H), lambda i, c: (c, 0, 0)),
        pl.BlockSpec((1, 1, H), lambda i, c: (c, 0, 0)),
        pl.BlockSpec((1, D, H), lambda i, c: (c, 0, 0)),
        pl.BlockSpec((1, D, H), lambda i, c: (c, 0, 0)),
    ],
    out_specs=[
        pl.BlockSpec((1, _RB, D), lambda i, c: (c, i, 0)),
        pl.BlockSpec((1, _RB, D), lambda i, c: (c, i, 0)),
    ],
    out_shape=[
        jax.ShapeDtypeStruct((NC, N, D), F32),
        jax.ShapeDtypeStruct((NC, N, D), F32),
    ],
)


def _k2_mid_body(acc_ref, fw_ref, fb_ref, fwh_ref, fbh_ref, wdh_ref, wsh_ref,
                 src2_ref, pd_ref):
  a0, a1 = acc_ref[0], acc_ref[1]
  h = jnp.concatenate(
      [a0[:, H:] / (a0[:, :H] + 1e-16), a1[:, H:] / (a1[:, :H] + 1e-16)],
      axis=1)
  t = jnp.dot(h, fw_ref[...], preferred_element_type=F32) + fb_ref[...]
  ps_h = jnp.dot(t, wsh_ref[0], preferred_element_type=F32)
  t_h = jnp.dot(h, fwh_ref[0], preferred_element_type=F32) + fbh_ref[0]
  src2_ref[0] = jnp.concatenate([ps_h, t_h], axis=1)
  pd_h = jnp.dot(t, wdh_ref[0], preferred_element_type=F32)
  pd_ref[0] = jnp.concatenate([pd_h, pd_h], axis=1)


_k2_mid = pl.pallas_call(
    _k2_mid_body,
    grid=(_NB, NC),
    in_specs=[
        pl.BlockSpec((NC, _RB, D), lambda i, c: (0, i, 0)),
        pl.BlockSpec((D, D), lambda i, c: (0, 0)),
        pl.BlockSpec((1, D), lambda i, c: (0, 0)),
        pl.BlockSpec((1, D, H), lambda i, c: (c, 0, 0)),
        pl.BlockSpec((1, 1, H), lambda i, c: (c, 0, 0)),
        pl.BlockSpec((1, D, H), lambda i, c: (c, 0, 0)),
        pl.BlockSpec((1, D, H), lambda i, c: (c, 0, 0)),
    ],
    out_specs=[
        pl.BlockSpec((1, _RB, D), lambda i, c: (c, i, 0)),
        pl.BlockSpec((1, _RB, D), lambda i, c: (c, i, 0)),
    ],
    out_shape=[
        jax.ShapeDtypeStruct((NC, N, D), F32),
        jax.ShapeDtypeStruct((NC, N, D), F32),
    ],
)


_EB = 2048
_NEB = E_PAD // _EB


def _k3_body(ea_ref, we_ref, be_ref, wa_ref, ab_ref, pe_ref):
  i = pl.program_id(0)
  m = jnp.dot(we_ref[...], wa_ref[...], preferred_element_type=F32)
  cvec = (jnp.dot(be_ref[...], wa_ref[...], preferred_element_type=F32)
          + ab_ref[...])
  pe = jnp.dot(ea_ref[...], m, preferred_element_type=F32) + cvec
  row = i * _EB + lax.broadcasted_iota(jnp.int32, (_EB, 1), 0)
  pe_ref[...] = jnp.where(row < E, pe, -1e30)


_k3 = pl.pallas_call(
    _k3_body,
    grid=(_NEB,),
    in_specs=[
        pl.BlockSpec((_EB, DE), lambda i: (i, 0)),
        pl.BlockSpec((DE, DE), lambda i: (0, 0)),
        pl.BlockSpec((1, DE), lambda i: (0, 0)),
        pl.BlockSpec((DE, D), lambda i: (0, 0)),
        pl.BlockSpec((1, D), lambda i: (0, 0)),
    ],
    out_specs=pl.BlockSpec((_EB, D), lambda i: (i, 0)),
    out_shape=jax.ShapeDtypeStruct((E_PAD, D), F32),
)


def _k4_body(acc_ref, out_ref):
  a0, a1 = acc_ref[0], acc_ref[1]
  out_ref[...] = jnp.concatenate(
      [a0[:, H:] / (a0[:, :H] + 1e-16), a1[:, H:] / (a1[:, :H] + 1e-16)],
      axis=1)


_k4 = pl.pallas_call(
    _k4_body,
    grid=(_NB,),
    in_specs=[
        pl.BlockSpec((NC, _RB, D), lambda i: (0, i, 0)),
    ],
    out_specs=pl.BlockSpec((_RB, D), lambda i: (i, 0)),
    out_shape=jax.ShapeDtypeStruct((N, D), F32),
)


@functools.cache
def _build_sc_edge_pass():
  sc_mesh = plsc.VectorSubcoreMesh(
      core_axis_name="c", subcore_axis_name="s", num_cores=NC,
      num_subcores=NS)

  @functools.partial(
      pl.kernel,
      out_type=jax.ShapeDtypeStruct((NC * N_ACC, D), F32),
      mesh=sc_mesh,
      scratch_types=[
          pltpu.VMEM((CHUNK,), jnp.int32),
          pltpu.VMEM((CHUNK,), jnp.int32),
          pltpu.VMEM((CHUNK,), jnp.int32),
          pltpu.VMEM((CHUNK,), jnp.int32),
          pltpu.VMEM((2 * CHUNK,), jnp.int32),
          pltpu.VMEM((2 * CHUNK,), jnp.int32),
          pltpu.VMEM((2 * CHUNK, D), F32),
          pltpu.VMEM((CHUNK, D), F32),
          pltpu.VMEM((2 * CHUNK, D), F32),
          pltpu.VMEM((CHUNK, D), F32),
          pltpu.VMEM((CHUNK,), jnp.int32),
          pltpu.VMEM((CHUNK,), jnp.int32),
          pltpu.VMEM_SHARED((N_ACC, D), F32),
          pltpu.SemaphoreType.DMA,
          pltpu.SemaphoreType.DMA,
          pltpu.SemaphoreType.DMA,
          pltpu.SemaphoreType.DMA,
          pltpu.SemaphoreType.DMA,
          pltpu.SemaphoreType.DMA,
      ],
  )
  def sc_edge_pass(src_hbm, dst_hbm, ct_hbm, pe_hbm,
                   acc_hbm,
                   isa, ida, isb, idb, cia, cib, cra, pea, crb, peb,
                   sda, sdb, acc, sema, semb, isema, isemb, ssema, ssemb):
    c = lax.axis_index("c")
    s = lax.axis_index("s")
    cH = c * H
    set_a = (isa, ida, cia, cra, pea, sema, isema, sda, ssema)
    set_b = (isb, idb, cib, crb, peb, semb, isemb, sdb, ssemb)

    zero = jnp.zeros((16,), F32)
    for r in range(8):
      for j in range(D // 16):
        cra[r, pl.ds(j * 16, 16)] = zero

    def zinit(k, carry):
      pltpu.sync_copy(cra.at[pl.ds(0, 8)],
                      acc.at[pl.ds(s * ROWS_PER_TILE + k * 8, 8)])
      return carry

    lax.fori_loop(0, ROWS_PER_TILE // 8, zinit, 0)
    plsc.subcore_barrier()

    def issue_idx(g, bufs):
      i_s, i_d = bufs[0], bufs[1]
      isem = bufs[6]
      base = s * EPT + g * CHUNK
      pltpu.async_copy(src_hbm.at[pl.ds(base, CHUNK)], i_s, isem)
      pltpu.async_copy(dst_hbm.at[pl.ds(base, CHUNK)], i_d, isem)

    def prefetch(g, bufs):
      i_s, i_d, ci, cr, pe_, sem, isem = bufs[:7]
      base = s * EPT + g * CHUNK
      pltpu.make_async_copy(src_hbm.at[pl.ds(0, CHUNK)], i_s, isem).wait()
      pltpu.make_async_copy(dst_hbm.at[pl.ds(0, CHUNK)], i_d, isem).wait()
      for j in range(CHUNK // 16):
        sl = pl.ds(j * 16, 16)
        ci[sl] = i_s[sl] + c * N
        ci[pl.ds(CHUNK + j * 16, 16)] = i_d[sl] + (2 * N + c * N)
      pltpu.async_copy(ct_hbm.at[ci], cr, sem)
      pltpu.async_copy(pe_hbm.at[pl.ds(base, CHUNK)], pe_, sem)

    def wait_set(bufs):
      i_s, i_d, ci, cr, pe_, sem = bufs[:6]
      pltpu.make_async_copy(ct_hbm.at[ci], cr, sem).wait()
      pltpu.make_async_copy(pe_hbm.at[pl.ds(0, CHUNK)], pe_, sem).wait()

    def wait_scatter(bufs):
      cr, sd, ssem = bufs[3], bufs[7], bufs[8]
      pltpu.make_async_copy(cr.at[pl.ds(0, CHUNK)], acc.at[sd], ssem).wait()

    def compute_scatter(bufs):
      i_s, i_d, ci, cr, pe_, sem, _, sd, ssem = bufs

      @plsc.parallel_loop(0, CHUNK, step=1, unroll=4)
      def edge_body(e):
        for j in range(H // 16):
          slh = pl.ds(cH + j * 16, 16)
          sl = pl.ds(j * 16, 16)
          slt = pl.ds(H + j * 16, 16)
          a = cr[CHUNK + e, sl] + cr[e, sl] + pe_[e, slh]
          a = jnp.maximum(a, a * 0.2)
          w = jnp.exp(a)
          tv = cr[e, slt]
          cr[e, sl] = w
          cr[e, slt] = w * tv
      for j in range(CHUNK // 16):
        sl = pl.ds(j * 16, 16)
        sd[sl] = i_d[sl]
      pltpu.async_copy(cr.at[pl.ds(0, CHUNK)], acc.at[sd], ssem, add=True)

    def pair_body(i, carry):
      g0 = 2 * i
      wait_set(set_a)

      @pl.when(i > 0)
      def _():
        wait_scatter(set_b)

      prefetch(g0 + 1, set_b)
      compute_scatter(set_a)

      @pl.when(g0 + 2 < NSTEP)
      def _():
        issue_idx(g0 + 2, set_a)

      wait_set(set_b)
      wait_scatter(set_a)

      @pl.when(g0 + 2 < NSTEP)
      def _():
        prefetch(g0 + 2, set_a)

      compute_scatter(set_b)

      @pl.when(g0 + 3 < NSTEP)
      def _():
        issue_idx(g0 + 3, set_b)

      return carry

    issue_idx(0, set_a)
    prefetch(0, set_a)
    issue_idx(1, set_b)
    lax.fori_loop(0, NSTEP // 2, pair_body, 0)
    wait_scatter(set_b)
    plsc.subcore_barrier()

    def flush(k, carry):
      r0 = s * ROWS_PER_TILE + k * 64
      pltpu.sync_copy(acc.at[pl.ds(r0, 64)], crb.at[pl.ds(0, 64)])
      pltpu.sync_copy(crb.at[pl.ds(0, 64)], acc_hbm.at[pl.ds(c * N_ACC + r0, 64)])
      return carry

    lax.fori_loop(0, ROWS_PER_TILE // 64, flush, 0)
    tail = ROWS_PER_TILE % 64
    if tail:
      r0 = s * ROWS_PER_TILE + (ROWS_PER_TILE // 64) * 64
      pltpu.sync_copy(acc.at[pl.ds(r0, tail)], crb.at[pl.ds(0, tail)])
      pltpu.sync_copy(crb.at[pl.ds(0, tail)],
                      acc_hbm.at[pl.ds(c * N_ACC + r0, tail)])

  return sc_edge_pass


def _split_cols(w):
  return w.reshape(D, NC, H).transpose(1, 0, 2)


def kernel(x, edge_index, edge_attr, W_node, b_node, W_edge, b_edge,
           gamma, beta, fc_W, fc_b, attn_W, attn_b):
  src = edge_index[0].astype(jnp.int32)
  dst = edge_index[1].astype(jnp.int32)
  src_pad = jnp.pad(src, (0, E_PAD - E))
  dst_pad = jnp.pad(dst, (0, E_PAD - E))
  ea_pad = jnp.pad(edge_attr.astype(F32), ((0, E_PAD - E), (0, 0)))

  h_pre, psum, psumsq = _k1(x, W_node, b_node.reshape(1, D))

  acc = None
  for l in range(3):
    aW = attn_W[l]
    wd, ws, wa = aW[:D], aW[D:2 * D], aW[2 * D:]
    fw = fc_W[l]
    fwh = _split_cols(fw)
    fbh = fc_b[l].reshape(NC, 1, H)
    wsh = _split_cols(ws)
    wdh = _split_cols(wd)
    if l == 0:
      src2, pdd = _k2_first(h_pre, psum, psumsq, gamma.reshape(1, D),
                            beta.reshape(1, D), fw, fc_b[0].reshape(1, D),
                            fwh, fbh, wdh, wsh)
    else:
      src2, pdd = _k2_mid(acc, fw, fc_b[l].reshape(1, D), fwh, fbh, wdh, wsh)
    pe = _k3(ea_pad, W_edge, b_edge.reshape(1, DE), wa,
             attn_b[l].reshape(1, D))
    ct = jnp.concatenate(
        [src2.reshape(NC * N, D), pdd.reshape(NC * N, D)], axis=0)
    acc = _build_sc_edge_pass()(src_pad, dst_pad, ct, pe)
    acc = acc.reshape(NC, N_ACC, D)[:, :N]

  h = _k4(acc)
  return h.reshape(1, N, D)
